# Initial kernel scaffold; baseline (speedup 1.0000x reference)
import jax
import jax.numpy as jnp
from jax.experimental import pallas as pl


def kernel(x, embedding):
    raise NotImplementedError("write your pallas kernel here")


# speedup vs baseline: 1.0935x; 1.0935x over previous
import functools

import jax
import jax.numpy as jnp
from jax import lax
from jax.experimental import pallas as pl
from jax.experimental.pallas import tpu as pltpu
from jax.experimental.pallas import tpu_sc as plsc

NUM_IDX = 16384 * 50
D = 32
NC = 2
NS = 16
NW = NC * NS
B_PER_W = NUM_IDX // NW
SUB = 128
CHUNK = 1024
N_SUB = CHUNK // SUB
N_CHUNKS = B_PER_W // CHUNK

_mesh = plsc.VectorSubcoreMesh(core_axis_name="c", subcore_axis_name="s")


@functools.partial(
    pl.kernel,
    mesh=_mesh,
    out_type=jax.ShapeDtypeStruct((NUM_IDX, D), jnp.float32),
    scratch_types=[
        pltpu.VMEM((CHUNK,), jnp.int32),
        pltpu.VMEM((CHUNK, D), jnp.float32),
        pltpu.SemaphoreType.DMA,
    ],
    compiler_params=pltpu.CompilerParams(use_tc_tiling_on_sc=False),
)
def _embed_sc(idx_hbm, table_hbm, out_hbm, idx_v, rows_v, sem):
    wid = lax.axis_index("s") * NC + lax.axis_index("c")
    base = wid * B_PER_W

    def body(i, _):
        off = base + i * CHUNK
        pltpu.sync_copy(idx_hbm.at[pl.ds(off, CHUNK)], idx_v)
        copies = [
            pltpu.async_copy(
                table_hbm.at[idx_v.at[pl.ds(j * SUB, SUB)]],
                rows_v.at[pl.ds(j * SUB, SUB)],
                sem,
            )
            for j in range(N_SUB)
        ]
        for c in copies:
            c.wait()
        pltpu.sync_copy(rows_v, out_hbm.at[pl.ds(off, CHUNK)])
        return ()

    lax.fori_loop(0, N_CHUNKS, body, ())


def kernel(x, embedding):
    idx = x.astype(jnp.int32).reshape(NUM_IDX)
    out = _embed_sc(idx, embedding)
    return out.reshape(x.shape[0], x.shape[1], D)


# speedup vs baseline: 1.0952x; 1.0016x over previous
import functools

import jax
import jax.numpy as jnp
from jax import lax
from jax.experimental import pallas as pl
from jax.experimental.pallas import tpu as pltpu
from jax.experimental.pallas import tpu_sc as plsc

NUM_IDX = 16384 * 50
D = 32
NC = 2
NS = 16
NW = NC * NS
B_PER_W = NUM_IDX // NW
SUB = 128
CHUNK = 1024
N_SUB = CHUNK // SUB
N_CHUNKS = B_PER_W // CHUNK

_mesh = plsc.VectorSubcoreMesh(core_axis_name="c", subcore_axis_name="s")


@functools.partial(
    pl.kernel,
    mesh=_mesh,
    out_type=jax.ShapeDtypeStruct((NUM_IDX, D), jnp.float32),
    scratch_types=[
        pltpu.VMEM((CHUNK,), jnp.int32),
        pltpu.VMEM((CHUNK, D), jnp.float32),
        pltpu.SemaphoreType.DMA,
    ],
    compiler_params=pltpu.CompilerParams(use_tc_tiling_on_sc=False),
)
def _embed_sc(idx_hbm, table_hbm, out_hbm, idx_v, rows_v, sem):
    wid = lax.axis_index("s") * NC + lax.axis_index("c")
    base = wid * B_PER_W

    def body(i, _):
        off = base + i * CHUNK
        pltpu.sync_copy(idx_hbm.at[pl.ds(off, CHUNK)], idx_v)
        pltpu.async_copy(table_hbm.at[idx_v], rows_v, sem).wait()
        pltpu.sync_copy(rows_v, out_hbm.at[pl.ds(off, CHUNK)])
        return ()

    lax.fori_loop(0, N_CHUNKS, body, ())


def kernel(x, embedding):
    idx = x.astype(jnp.int32).reshape(NUM_IDX)
    out = _embed_sc(idx, embedding)
    return out.reshape(x.shape[0], x.shape[1], D)


# speedup vs baseline: 1.7046x; 1.5565x over previous
import functools

import jax
import jax.numpy as jnp
from jax import lax
from jax.experimental import pallas as pl
from jax.experimental.pallas import tpu as pltpu
from jax.experimental.pallas import tpu_sc as plsc

B, S = 16384, 50
NUM_IDX = B * S
D = 32
NC = 2
NS = 16
NW = NC * NS
ROWS_PER_W = B // NW
CHUNK_ROWS = 16
CHUNK = CHUNK_ROWS * S
N_CHUNKS = ROWS_PER_W // CHUNK_ROWS

_mesh = plsc.VectorSubcoreMesh(core_axis_name="c", subcore_axis_name="s")


@functools.partial(
    pl.kernel,
    mesh=_mesh,
    out_type=jax.ShapeDtypeStruct((B, S, D), jnp.float32),
    scratch_types=[
        pltpu.VMEM((CHUNK,), jnp.int32),
        pltpu.VMEM((CHUNK, D), jnp.float32),
        pltpu.SemaphoreType.DMA,
    ],
    compiler_params=pltpu.CompilerParams(use_tc_tiling_on_sc=False),
)
def _embed_sc(idx_hbm, table_hbm, out_hbm, idx_v, rows_v, sem):
    wid = lax.axis_index("s") * NC + lax.axis_index("c")
    base_row = wid * ROWS_PER_W

    def body(i, _):
        r0 = base_row + i * CHUNK_ROWS
        pltpu.sync_copy(idx_hbm.at[pl.ds(r0 * S, CHUNK)], idx_v)
        pltpu.async_copy(table_hbm.at[idx_v], rows_v, sem).wait()
        for k in range(CHUNK_ROWS):
            pltpu.sync_copy(rows_v.at[pl.ds(k * S, S)], out_hbm.at[r0 + k])
        return ()

    lax.fori_loop(0, N_CHUNKS, body, ())


def kernel(x, embedding):
    idx = x.astype(jnp.int32).reshape(NUM_IDX)
    return _embed_sc(idx, embedding)
